# baseline (speedup 1.0000x reference)
import functools

import jax
import jax.numpy as jnp
from jax import lax
from jax.experimental import pallas as pl
from jax.experimental.pallas import tpu as pltpu
from jax.experimental.pallas import tpu_sc as plsc

D = 64
CW = 128
L = 16


def _build(B, dtype):
    info = plsc.get_sparse_core_info()
    NC, NS = info.num_cores, info.num_subcores
    NW = NC * NS
    BW = B // NW
    P = BW // CW

    mesh = plsc.VectorSubcoreMesh(core_axis_name="c", subcore_axis_name="s")

    @functools.partial(
        pl.kernel,
        mesh=mesh,
        out_type=jax.ShapeDtypeStruct((B, 2, D), dtype),
        compiler_params=pltpu.CompilerParams(use_tc_tiling_on_sc=False),
        scratch_types=[
            pltpu.VMEM((P, CW), jnp.int32),
            pltpu.VMEM((P, CW), jnp.int32),
            pltpu.VMEM((P, CW), jnp.int32),
            pltpu.VMEM((CW, D), dtype),
            pltpu.VMEM((CW, D), dtype),
            pltpu.VMEM((CW, D), dtype),
            pltpu.VMEM((CW, 2, D), dtype),
            pltpu.SemaphoreType.DMA,
        ],
    )
    def k(h_hbm, r_hbm, t_hbm, head_hbm, rel_hbm, rat_hbm, out_hbm,
          hi, ri, ti, hb, eb, ab, ob, sem):
        wid = lax.axis_index("s") * NC + lax.axis_index("c")
        base = wid * BW
        row0 = wid * P
        pltpu.sync_copy(h_hbm.at[pl.ds(row0, P)], hi)
        pltpu.sync_copy(r_hbm.at[pl.ds(row0, P)], ri)
        pltpu.sync_copy(t_hbm.at[pl.ds(row0, P)], ti)
        for p in range(P):
            c0 = pltpu.async_copy(head_hbm.at[hi.at[p]], hb, sem)
            c1 = pltpu.async_copy(rel_hbm.at[ri.at[p]], eb, sem)
            c2 = pltpu.async_copy(rat_hbm.at[ti.at[p]], ab, sem)
            c0.wait()
            c1.wait()
            c2.wait()

            def row(i, carry):
                for dd in range(D // L):
                    sl = pl.ds(dd * L, L)
                    av = ab[i, sl]
                    ob[i, 0, sl] = hb[i, sl] + av
                    ob[i, 1, sl] = eb[i, sl] + av
                return carry

            lax.fori_loop(0, CW, row, 0)
            pltpu.sync_copy(ob, out_hbm.at[pl.ds(base + p * CW, CW)])

    return k


def kernel(h, r, t, head_table, relation_table, rating_table):
    B = h.shape[0]
    h2 = h.reshape(B // CW, CW).astype(jnp.int32)
    r2 = r.reshape(B // CW, CW).astype(jnp.int32)
    t2 = t.reshape(B // CW, CW).astype(jnp.int32)
    k = _build(B, head_table.dtype)
    return k(h2, r2, t2, head_table, relation_table, rating_table)


# speedup vs baseline: 2.1218x; 2.1218x over previous
import functools

import jax
import jax.numpy as jnp
from jax import lax
from jax.experimental import pallas as pl
from jax.experimental.pallas import tpu as pltpu
from jax.experimental.pallas import tpu_sc as plsc

D = 64
CW = 64
L = 16


def _build(B):
    info = plsc.get_sparse_core_info()
    NC, NS = info.num_cores, info.num_subcores
    NW = NC * NS
    BW = B // NW
    P = BW // CW
    row_bytes = D * 4

    mesh = plsc.VectorSubcoreMesh(core_axis_name="c", subcore_axis_name="s")

    @functools.partial(
        pl.kernel,
        mesh=mesh,
        out_type=jax.ShapeDtypeStruct((B, 2 * D), jnp.float32),
        scratch_types=[
            pltpu.VMEM((BW,), jnp.int32),
            pltpu.VMEM((BW,), jnp.int32),
            pltpu.VMEM((BW,), jnp.int32),
            pltpu.VMEM((3, D), jnp.float32),
            pltpu.VMEM((5, D), jnp.float32),
            pltpu.VMEM((15, D), jnp.float32),
            pltpu.VMEM((2, CW, D), jnp.float32),
            pltpu.VMEM((2, CW, 2 * D), jnp.float32),
            pltpu.SemaphoreType.DMA,
            pltpu.SemaphoreType.DMA,
            pltpu.SemaphoreType.DMA,
        ],
    )
    def k(h_hbm, r_hbm, t_hbm, head_hbm, rel_hbm, rat_hbm, out_hbm,
          hi, ri, ti, rel_t, rat_t, sum_t, rows, ob, sem0, sem1, sem_o):
        wid = lax.axis_index("s") * NC + lax.axis_index("c")
        base = wid * BW
        pltpu.sync_copy(h_hbm.at[wid], hi)
        pltpu.sync_copy(r_hbm.at[wid], ri)
        pltpu.sync_copy(t_hbm.at[wid], ti)
        pltpu.sync_copy(rel_hbm, rel_t)
        pltpu.sync_copy(rat_hbm, rat_t)

        for rr in range(3):
            for tt in range(5):
                for dd in range(D // L):
                    sl = pl.ds(dd * L, L)
                    sum_t[rr * 5 + tt, sl] = rel_t[rr, sl] + rat_t[tt, sl]

        sems = (sem0, sem1)

        def issue(p, b):
            def chunk(g, carry):
                hvec = hi[pl.ds(p * CW + g * L, L)]
                for il in range(L):
                    pltpu.async_copy(
                        head_hbm.at[hvec[il]],
                        rows.at[b, g * L + il],
                        sems[b],
                    )
                return carry

            lax.fori_loop(0, CW // L, chunk, 0)

        def drain(b):
            for _ in range(CW):
                pltpu.make_async_copy(
                    head_hbm.at[0], rows.at[b, 0], sems[b]
                ).wait()

        def compute(p, b):
            def chunk(g, carry):
                off = p * CW + g * L
                tvec = ti[pl.ds(off, L)]
                cvec = ri[pl.ds(off, L)] * 5 + tvec
                for il in range(L):
                    i = g * L + il
                    t_s = tvec[il]
                    c_s = cvec[il]
                    for dd in range(D // L):
                        sl = pl.ds(dd * L, L)
                        ob[b, i, sl] = rows[b, i, sl] + rat_t[t_s, sl]
                        ob[b, i, pl.ds(D + dd * L, L)] = sum_t[c_s, sl]
                return carry

            lax.fori_loop(0, CW // L, chunk, 0)

        stores = [None, None]
        issue(0, 0)
        for p in range(P):
            b = p % 2
            drain(b)
            if p + 1 < P:
                issue(p + 1, 1 - b)
            if stores[b] is not None:
                stores[b].wait()
            compute(p, b)
            stores[b] = pltpu.async_copy(
                ob.at[b], out_hbm.at[pl.ds(base + p * CW, CW)], sem_o)
        for st in stores:
            if st is not None:
                st.wait()

    return k


def kernel(h, r, t, head_table, relation_table, rating_table):
    B = h.shape[0]
    info = plsc.get_sparse_core_info()
    NW = info.num_cores * info.num_subcores
    h2 = h.reshape(NW, B // NW).astype(jnp.int32)
    r2 = r.reshape(NW, B // NW).astype(jnp.int32)
    t2 = t.reshape(NW, B // NW).astype(jnp.int32)
    k = _build(B)
    packed = k(h2, r2, t2, head_table, relation_table, rating_table)
    return packed.reshape(B, 2, D)
